# baseline (speedup 1.0000x reference)
import functools
import math

import jax
import jax.numpy as jnp
from jax import lax
from jax.experimental import pallas as pl
from jax.experimental.pallas import tpu as pltpu
from jax.experimental.pallas import tpu_sc as plsc

_E = 16
_D = 2048
_BLK = 1024
_NW = 32
_N = 8192
_CHUNK = _N // _NW
_ROWS = _CHUNK // 128


def _router_body(x_ref, W_ref, b_ref, probs_ref, a_ref, d_ref, acc_ref):
    i = pl.program_id(0)
    logits = jax.lax.dot_general(
        W_ref[...], x_ref[...],
        dimension_numbers=(((1,), (1,)), ((), ())),
        preferred_element_type=jnp.float32,
    ) + b_ref[...]
    m = jnp.max(logits, axis=0, keepdims=True)
    e = jnp.exp(logits - m)
    s = jnp.sum(e, axis=0, keepdims=True)
    probs_ref[...] = jnp.transpose(e / s, (1, 0))
    lse = m + jnp.log(s)
    half0 = jnp.sum(logits[: _E // 2, :], axis=0, keepdims=True)
    half1 = jnp.sum(logits[_E // 2 :, :], axis=0, keepdims=True)
    d_ref[...] = jnp.reshape(half1 - half0, (_BLK // 128, 128))
    part = jnp.sum(lse - 0.125 * half0)

    @pl.when(i == 0)
    def _init():
        acc_ref[0] = 0.0

    acc_ref[0] += part

    @pl.when(i == pl.num_programs(0) - 1)
    def _fin():
        a_ref[0, 0] = acc_ref[0]


def _kl_sc_body(bits_hbm, d_hbm, out_hbm, bits_v, d_v, part_v):
    wid = lax.axis_index("s") * 2 + lax.axis_index("c")
    base = wid * _ROWS
    pltpu.sync_copy(bits_hbm.at[pl.ds(base, _ROWS), :], bits_v)
    pltpu.sync_copy(d_hbm.at[pl.ds(base, _ROWS), :], d_v)
    acc = jnp.zeros((16,), jnp.float32)
    for j in range(_CHUNK // 16):
        r, col = divmod(j * 16, 128)
        bv = bits_v[r, pl.ds(col, 16)]
        dv = d_v[r, pl.ds(col, 16)]
        acc = acc + jnp.where(bv == 1, dv, jnp.float32(0.0))
    part_v[...] = acc
    pltpu.sync_copy(part_v, out_hbm.at[wid])


@jax.jit
def kernel(x, bits, W, b):
    n = x.shape[0]
    nblk = n // _BLK
    bits2 = bits.astype(jnp.int32).reshape(n // 128, 128)
    b2 = b.astype(jnp.float32).reshape(_E, 1)
    probs, a, d64 = pl.pallas_call(
        _router_body,
        grid=(nblk,),
        in_specs=[
            pl.BlockSpec((_BLK, _D), lambda i: (i, 0)),
            pl.BlockSpec((_E, _D), lambda i: (0, 0)),
            pl.BlockSpec((_E, 1), lambda i: (0, 0)),
        ],
        out_specs=[
            pl.BlockSpec((_BLK, _E), lambda i: (i, 0)),
            pl.BlockSpec(memory_space=pltpu.SMEM),
            pl.BlockSpec((_BLK // 128, 128), lambda i: (i, 0)),
        ],
        out_shape=[
            jax.ShapeDtypeStruct((n, _E), jnp.float32),
            jax.ShapeDtypeStruct((1, 1), jnp.float32),
            jax.ShapeDtypeStruct((n // 128, 128), jnp.float32),
        ],
        scratch_shapes=[pltpu.SMEM((1,), jnp.float32)],
    )(x, W, b2)

    sc_kl = functools.partial(
        pl.kernel,
        mesh=plsc.VectorSubcoreMesh(core_axis_name="c", subcore_axis_name="s"),
        out_type=jax.ShapeDtypeStruct((_NW, 16), jnp.float32),
        scratch_types=[
            pltpu.VMEM((_ROWS, 128), jnp.int32),
            pltpu.VMEM((_ROWS, 128), jnp.float32),
            pltpu.VMEM((16,), jnp.float32),
        ],
    )(_kl_sc_body)
    parts = sc_kl(bits2, d64)

    kl = (a[0, 0] - 0.125 * jnp.sum(parts)) / n - math.log(8.0)
    return (probs, kl)


# speedup vs baseline: 1.6383x; 1.6383x over previous
import jax
import jax.numpy as jnp
from jax.experimental import pallas as pl
from jax.experimental.pallas import tpu as pltpu

_E = 16
_D = 2048
_BLK = 1024


def _router_body(x_ref, bits_ref, W_ref, b_ref, probs_ref, kl_ref, acc_ref):
    i = pl.program_id(0)
    logits = jax.lax.dot_general(
        W_ref[...], x_ref[...],
        dimension_numbers=(((1,), (1,)), ((), ())),
        preferred_element_type=jnp.float32,
    ) + b_ref[...]
    e = jnp.exp(logits)
    s = jnp.sum(e, axis=0, keepdims=True)
    probs_ref[...] = jnp.transpose(e / s, (1, 0))
    lse = jnp.log(s)
    half0 = jnp.sum(logits[: _E // 2, :], axis=0, keepdims=True)
    half1 = jnp.sum(logits[_E // 2 :, :], axis=0, keepdims=True)
    bsel = bits_ref[...].astype(jnp.float32).reshape(1, _BLK)
    halfsum = half0 + bsel * (half1 - half0)
    part = jnp.sum(lse - 0.125 * halfsum)

    @pl.when(i == 0)
    def _init():
        acc_ref[0] = 0.0

    acc_ref[0] += part

    @pl.when(i == pl.num_programs(0) - 1)
    def _fin():
        n = _BLK * pl.num_programs(0)
        kl_ref[0, 0] = acc_ref[0] / n - jnp.log(jnp.float32(8.0))


@jax.jit
def kernel(x, bits, W, b):
    n = x.shape[0]
    nblk = n // _BLK
    bits2 = bits.astype(jnp.int32).reshape(n // 128, 128)
    b2 = b.astype(jnp.float32).reshape(_E, 1)
    probs, kl = pl.pallas_call(
        _router_body,
        grid=(nblk,),
        in_specs=[
            pl.BlockSpec((_BLK, _D), lambda i: (i, 0)),
            pl.BlockSpec((_BLK // 128, 128), lambda i: (i, 0)),
            pl.BlockSpec((_E, _D), lambda i: (0, 0)),
            pl.BlockSpec((_E, 1), lambda i: (0, 0)),
        ],
        out_specs=[
            pl.BlockSpec((_BLK, _E), lambda i: (i, 0)),
            pl.BlockSpec(memory_space=pltpu.SMEM),
        ],
        out_shape=[
            jax.ShapeDtypeStruct((n, _E), jnp.float32),
            jax.ShapeDtypeStruct((1, 1), jnp.float32),
        ],
        scratch_shapes=[pltpu.SMEM((1,), jnp.float32)],
    )(x, bits2, W, b2)
    return (probs, kl.reshape(()))
